# baseline (speedup 1.0000x reference)
import functools

import jax
import jax.numpy as jnp
from jax import lax
from jax.experimental import pallas as pl
from jax.experimental.pallas import tpu as pltpu
from jax.experimental.pallas import tpu_sc as plsc

NC = 2
NS = 16
NW = NC * NS
LANES = 16
C = 80


def _sc_body(nch, s_cnt, d, idx_hbm, feat_hbm, self_hbm, nsum_hbm,
             idx_v, sbufs_v, abufs_v, sem_i, sem_sg, sem_ag, sem_sw,
             sem_aw):
    cid = lax.axis_index("c")
    sid = lax.axis_index("s")
    wid = sid * NC + (1 - cid)
    base = wid * (C * nch)
    rows_per_chunk = 1 + s_cnt

    pltpu.sync_copy(idx_hbm.at[wid], idx_v)

    self_g = []
    acc_g = []
    for k in range(nch):
        r0 = k * rows_per_chunk
        self_g.append(pltpu.async_copy(
            feat_hbm.at[idx_v.at[r0]], sbufs_v.at[k], sem_sg.at[k]))
        acc_g.append(pltpu.async_copy(
            feat_hbm.at[idx_v.at[r0 + 1]], abufs_v.at[k], sem_ag.at[k]))
    add_cp = []
    for k in range(nch):
        r0 = k * rows_per_chunk
        acc_g[k].wait()
        for s in range(1, s_cnt):
            add_cp.append(pltpu.async_copy(
                feat_hbm.at[idx_v.at[r0 + 1 + s]], abufs_v.at[k],
                sem_ag.at[k], add=True))
    self_w = []
    acc_w = []
    for k in range(nch):
        b0 = base + k * C
        self_g[k].wait()
        self_w.append(pltpu.async_copy(
            sbufs_v.at[k], self_hbm.at[pl.ds(b0, C)], sem_sw.at[k]))
    for k in range(nch):
        for s in range(1, s_cnt):
            add_cp[k * (s_cnt - 1) + s - 1].wait()
        b0 = base + k * C
        acc_w.append(pltpu.async_copy(
            abufs_v.at[k], nsum_hbm.at[pl.ds(b0, C)], sem_aw.at[k]))
    for k in range(nch):
        self_w[k].wait()
        acc_w[k].wait()


def _tc_body(inv_s, d, s_ref, n_ref, w_ref, o_ref):
    w = w_ref[...]
    w1 = w[:, :d]
    w2 = w[:, d:]
    x = lax.dot_general(s_ref[...], w1, (((1,), (1,)), ((), ())),
                        preferred_element_type=jnp.float32)
    y = lax.dot_general(n_ref[...], w2, (((1,), (1,)), ((), ())),
                        preferred_element_type=jnp.float32)
    o_ref[...] = jnp.maximum(x + y * inv_s, 0.0)


def kernel(nodes, features, neigh_idx, W):
    b_cnt, s_cnt = neigh_idx.shape
    n_nodes, d = features.shape
    nch = -(-b_cnt // (NW * C))
    bp = NW * C * nch
    pad = bp - b_cnt

    nodes_p = jnp.pad(nodes.astype(jnp.int32), (0, pad))
    neigh_p = jnp.pad(neigh_idx.astype(jnp.int32), ((0, pad), (0, 0)))
    self_part = nodes_p.reshape(NW, nch, 1, C)
    neigh_part = neigh_p.reshape(NW, nch, C, s_cnt).transpose(0, 1, 3, 2)
    idx_all = jnp.concatenate([self_part, neigh_part], axis=2)
    idx_all = idx_all.reshape(NW, nch * (1 + s_cnt), C)

    mesh = plsc.VectorSubcoreMesh(core_axis_name="c", subcore_axis_name="s")
    sc = pl.kernel(
        functools.partial(_sc_body, nch, s_cnt, d),
        out_type=[
            jax.ShapeDtypeStruct((bp, d), jnp.float32),
            jax.ShapeDtypeStruct((bp, d), jnp.float32),
        ],
        mesh=mesh,
        scratch_types=[
            pltpu.VMEM((nch * (1 + s_cnt), C), jnp.int32),
            pltpu.VMEM((nch, C, d), jnp.float32),
            pltpu.VMEM((nch, C, d), jnp.float32),
            pltpu.SemaphoreType.DMA,
            pltpu.SemaphoreType.DMA((nch,)),
            pltpu.SemaphoreType.DMA((nch,)),
            pltpu.SemaphoreType.DMA((nch,)),
            pltpu.SemaphoreType.DMA((nch,)),
        ],
    )
    self_rows, nsum = sc(idx_all, features)

    bm = 512
    out = pl.pallas_call(
        functools.partial(_tc_body, 1.0 / s_cnt, d),
        grid=(bp // bm,),
        in_specs=[
            pl.BlockSpec((bm, d), lambda i: (i, 0)),
            pl.BlockSpec((bm, d), lambda i: (i, 0)),
            pl.BlockSpec((d, 2 * d), lambda i: (0, 0)),
        ],
        out_specs=pl.BlockSpec((bm, d), lambda i: (i, 0)),
        out_shape=jax.ShapeDtypeStruct((bp, d), jnp.float32),
    )(self_rows, nsum, W)
    return out[:b_cnt]


# speedup vs baseline: 2.9112x; 2.9112x over previous
import functools

import jax
import jax.numpy as jnp
from jax import lax
from jax.experimental import pallas as pl
from jax.experimental.pallas import tpu as pltpu
from jax.experimental.pallas import tpu_sc as plsc

NC = 2
NS = 16
NW = NC * NS
LANES = 16
C = 80


def _sc_body(nch, s_cnt, d, idx_hbm, feat_hbm, self_hbm, nsum_hbm,
             idx_v, sbufs_v, abufs_v, sem_i, sem_sg, sem_ag, sem_sw,
             sem_aw):
    cid = lax.axis_index("c")
    sid = lax.axis_index("s")
    wid = sid * NC + cid
    base = wid * (C * nch)
    rows_per_chunk = 1 + s_cnt

    pltpu.sync_copy(idx_hbm.at[wid], idx_v)

    self_g = []
    acc_g = []
    for k in range(nch):
        r0 = k * rows_per_chunk
        self_g.append(pltpu.async_copy(
            feat_hbm.at[idx_v.at[r0]], sbufs_v.at[k], sem_sg.at[k]))
        acc_g.append(pltpu.async_copy(
            feat_hbm.at[idx_v.at[r0 + 1]], abufs_v.at[k], sem_ag.at[k]))
    add_cp = []
    for k in range(nch):
        r0 = k * rows_per_chunk
        acc_g[k].wait()
        for s in range(1, s_cnt):
            add_cp.append(pltpu.async_copy(
                feat_hbm.at[idx_v.at[r0 + 1 + s]], abufs_v.at[k],
                sem_ag.at[k], add=True))
    self_w = []
    acc_w = []
    for k in range(nch):
        b0 = base + k * C
        self_g[k].wait()
        self_w.append(pltpu.async_copy(
            sbufs_v.at[k], self_hbm.at[pl.ds(b0, C)], sem_sw.at[k]))
    for k in range(nch):
        for s in range(1, s_cnt):
            add_cp[k * (s_cnt - 1) + s - 1].wait()
        b0 = base + k * C
        acc_w.append(pltpu.async_copy(
            abufs_v.at[k], nsum_hbm.at[pl.ds(b0, C)], sem_aw.at[k]))
    for k in range(nch):
        self_w[k].wait()
        acc_w[k].wait()


def _tc_body(inv_s, d, s_ref, n_ref, w_ref, o_ref):
    w = w_ref[...]
    w1 = w[:, :d]
    w2 = w[:, d:]
    x = lax.dot_general(s_ref[...], w1, (((1,), (1,)), ((), ())),
                        preferred_element_type=jnp.float32)
    y = lax.dot_general(n_ref[...], w2, (((1,), (1,)), ((), ())),
                        preferred_element_type=jnp.float32)
    o_ref[...] = jnp.maximum(x + y * inv_s, 0.0)


def kernel(nodes, features, neigh_idx, W):
    b_cnt, s_cnt = neigh_idx.shape
    n_nodes, d = features.shape
    nch = -(-b_cnt // (NW * C))
    bp = NW * C * nch
    pad = bp - b_cnt

    pad_nodes = jnp.arange(pad, dtype=jnp.int32) % n_nodes
    pad_neigh = (jnp.arange(pad * s_cnt, dtype=jnp.int32)
                 % n_nodes).reshape(pad, s_cnt)
    nodes_p = jnp.concatenate([nodes.astype(jnp.int32), pad_nodes])
    neigh_p = jnp.concatenate(
        [neigh_idx.astype(jnp.int32), pad_neigh], axis=0)
    self_part = nodes_p.reshape(NW, nch, 1, C)
    neigh_part = neigh_p.reshape(NW, nch, C, s_cnt).transpose(0, 1, 3, 2)
    idx_all = jnp.concatenate([self_part, neigh_part], axis=2)
    idx_all = idx_all.reshape(NW, nch * (1 + s_cnt), C)

    mesh = plsc.VectorSubcoreMesh(core_axis_name="c", subcore_axis_name="s")
    sc = pl.kernel(
        functools.partial(_sc_body, nch, s_cnt, d),
        out_type=[
            jax.ShapeDtypeStruct((bp, d), jnp.float32),
            jax.ShapeDtypeStruct((bp, d), jnp.float32),
        ],
        mesh=mesh,
        scratch_types=[
            pltpu.VMEM((nch * (1 + s_cnt), C), jnp.int32),
            pltpu.VMEM((nch, C, d), jnp.float32),
            pltpu.VMEM((nch, C, d), jnp.float32),
            pltpu.SemaphoreType.DMA,
            pltpu.SemaphoreType.DMA((nch,)),
            pltpu.SemaphoreType.DMA((nch,)),
            pltpu.SemaphoreType.DMA((nch,)),
            pltpu.SemaphoreType.DMA((nch,)),
        ],
    )
    self_rows, nsum = sc(idx_all, features)

    bm = 512
    out = pl.pallas_call(
        functools.partial(_tc_body, 1.0 / s_cnt, d),
        grid=(bp // bm,),
        in_specs=[
            pl.BlockSpec((bm, d), lambda i: (i, 0)),
            pl.BlockSpec((bm, d), lambda i: (i, 0)),
            pl.BlockSpec((d, 2 * d), lambda i: (0, 0)),
        ],
        out_specs=pl.BlockSpec((bm, d), lambda i: (i, 0)),
        out_shape=jax.ShapeDtypeStruct((bp, d), jnp.float32),
    )(self_rows, nsum, W)
    return out[:b_cnt]


# speedup vs baseline: 3.3850x; 1.1628x over previous
import functools

import jax
import jax.numpy as jnp
from jax import lax
from jax.experimental import pallas as pl
from jax.experimental.pallas import tpu as pltpu
from jax.experimental.pallas import tpu_sc as plsc

NC = 2
NS = 16
NW = NC * NS
LANES = 16
C = 80


def _sc_body(nch, s_cnt, d, idx_hbm, feat_hbm, self_hbm, nsum_hbm,
             idx_v, sbufs_v, abufs_v, sem_i, sem_sg, sem_ag, sem_sw,
             sem_aw):
    cid = lax.axis_index("c")
    sid = lax.axis_index("s")
    wid = sid * NC + cid
    base = wid * (C * nch)
    rows_per_chunk = 1 + s_cnt

    pltpu.sync_copy(idx_hbm.at[wid], idx_v)

    self_g = []
    acc_g = []
    for k in range(nch):
        r0 = k * rows_per_chunk
        self_g.append(pltpu.async_copy(
            feat_hbm.at[idx_v.at[r0]], sbufs_v.at[k], sem_sg.at[k]))
        acc_g.append(pltpu.async_copy(
            feat_hbm.at[idx_v.at[r0 + 1]], abufs_v.at[k], sem_ag.at[k]))
    add_cp = []
    for k in range(nch):
        r0 = k * rows_per_chunk
        acc_g[k].wait()
        for s in range(1, s_cnt):
            add_cp.append(pltpu.async_copy(
                feat_hbm.at[idx_v.at[r0 + 1 + s]], abufs_v.at[k],
                sem_ag.at[k], add=True))
    self_w = []
    acc_w = []
    for k in range(nch):
        b0 = base + k * C
        self_g[k].wait()
        self_w.append(pltpu.async_copy(
            sbufs_v.at[k], self_hbm.at[pl.ds(b0, C)], sem_sw.at[k]))
    for k in range(nch):
        for s in range(1, s_cnt):
            add_cp[k * (s_cnt - 1) + s - 1].wait()
        b0 = base + k * C
        acc_w.append(pltpu.async_copy(
            abufs_v.at[k], nsum_hbm.at[pl.ds(b0, C)], sem_aw.at[k]))
    for k in range(nch):
        self_w[k].wait()
        acc_w[k].wait()


def _tc_body(inv_s, d, s_ref, n_ref, w_ref, o_ref):
    w = w_ref[...]
    w1 = w[:, :d]
    w2 = w[:, d:]
    x = lax.dot_general(s_ref[...], w1, (((1,), (1,)), ((), ())),
                        preferred_element_type=jnp.float32)
    y = lax.dot_general(n_ref[...], w2, (((1,), (1,)), ((), ())),
                        preferred_element_type=jnp.float32)
    o_ref[...] = jnp.maximum(x + y * inv_s, 0.0)


def kernel(nodes, features, neigh_idx, W):
    b_cnt, s_cnt = neigh_idx.shape
    n_nodes, d = features.shape
    nch = -(-b_cnt // (NW * C))
    bp = NW * C * nch
    pad = bp - b_cnt

    pad_nodes = jnp.arange(pad, dtype=jnp.int32) % n_nodes
    pad_neigh = (jnp.arange(pad * s_cnt, dtype=jnp.int32)
                 % n_nodes).reshape(pad, s_cnt)
    nodes_p = jnp.concatenate([nodes.astype(jnp.int32), pad_nodes])
    neigh_p = jnp.concatenate(
        [neigh_idx.astype(jnp.int32), pad_neigh], axis=0)
    self_part = nodes_p.reshape(NW, nch, 1, C)
    neigh_part = neigh_p.reshape(NW, nch, C, s_cnt).transpose(0, 1, 3, 2)
    idx_all = jnp.concatenate([self_part, neigh_part], axis=2)
    idx_all = idx_all.reshape(NW, nch * (1 + s_cnt), C)

    mesh = plsc.VectorSubcoreMesh(core_axis_name="c", subcore_axis_name="s")
    sc = pl.kernel(
        functools.partial(_sc_body, nch, s_cnt, d),
        out_type=[
            jax.ShapeDtypeStruct((bp, d), jnp.float32),
            jax.ShapeDtypeStruct((bp, d), jnp.float32),
        ],
        mesh=mesh,
        scratch_types=[
            pltpu.VMEM((nch * (1 + s_cnt), C), jnp.int32),
            pltpu.VMEM((nch, C, d), jnp.float32),
            pltpu.VMEM((nch, C, d), jnp.float32),
            pltpu.SemaphoreType.DMA,
            pltpu.SemaphoreType.DMA((nch,)),
            pltpu.SemaphoreType.DMA((nch,)),
            pltpu.SemaphoreType.DMA((nch,)),
            pltpu.SemaphoreType.DMA((nch,)),
        ],
    )
    self_rows, nsum = sc(idx_all, features)

    bm = 1024
    out = pl.pallas_call(
        functools.partial(_tc_body, 1.0 / s_cnt, d),
        grid=(bp // bm,),
        in_specs=[
            pl.BlockSpec((bm, d), lambda i: (i, 0)),
            pl.BlockSpec((bm, d), lambda i: (i, 0)),
            pl.BlockSpec((d, 2 * d), lambda i: (0, 0)),
        ],
        out_specs=pl.BlockSpec((bm, d), lambda i: (i, 0)),
        out_shape=jax.ShapeDtypeStruct((b_cnt, d), jnp.float32),
    )(self_rows, nsum, W)
    return out


# speedup vs baseline: 3.5259x; 1.0416x over previous
import functools

import jax
import jax.numpy as jnp
from jax import lax
from jax.experimental import pallas as pl
from jax.experimental.pallas import tpu as pltpu
from jax.experimental.pallas import tpu_sc as plsc

NC = 2
NS = 16
NW = NC * NS
LANES = 16
C = 80


def _sc_body(nch, s_cnt, d, idx_hbm, feat_hbm, self_hbm, nsum_hbm,
             idx_v, sbufs_v, abufs_v, sem_i, sem_sg, sem_ag, sem_sw,
             sem_aw):
    cid = lax.axis_index("c")
    sid = lax.axis_index("s")
    wid = sid * NC + cid
    base = wid * (C * nch)
    rows_per_chunk = 1 + s_cnt

    pltpu.sync_copy(idx_hbm.at[wid], idx_v)

    self_g = []
    acc_g = []
    for k in range(nch):
        r0 = k * rows_per_chunk
        self_g.append(pltpu.async_copy(
            feat_hbm.at[idx_v.at[r0]], sbufs_v.at[k], sem_sg.at[k]))
        acc_g.append(pltpu.async_copy(
            feat_hbm.at[idx_v.at[r0 + 1]], abufs_v.at[k], sem_ag.at[k]))
    add_cp = []
    for k in range(nch):
        r0 = k * rows_per_chunk
        acc_g[k].wait()
        for s in range(1, s_cnt):
            add_cp.append(pltpu.async_copy(
                feat_hbm.at[idx_v.at[r0 + 1 + s]], abufs_v.at[k],
                sem_ag.at[k], add=True))
    self_w = []
    acc_w = []
    for k in range(nch):
        b0 = base + k * C
        self_g[k].wait()
        self_w.append(pltpu.async_copy(
            sbufs_v.at[k], self_hbm.at[pl.ds(b0, C)], sem_sw.at[k]))
    for k in range(nch):
        for s in range(1, s_cnt):
            add_cp[k * (s_cnt - 1) + s - 1].wait()
        b0 = base + k * C
        acc_w.append(pltpu.async_copy(
            abufs_v.at[k], nsum_hbm.at[pl.ds(b0, C)], sem_aw.at[k]))
    for k in range(nch):
        self_w[k].wait()
        acc_w[k].wait()


def _tc_body(inv_s, d, s_ref, n_ref, w_ref, o_ref):
    x = jnp.concatenate([s_ref[...], n_ref[...] * inv_s], axis=1)
    y = lax.dot_general(x, w_ref[...], (((1,), (1,)), ((), ())),
                        preferred_element_type=jnp.float32)
    o_ref[...] = jnp.maximum(y, 0.0)


def kernel(nodes, features, neigh_idx, W):
    b_cnt, s_cnt = neigh_idx.shape
    n_nodes, d = features.shape
    nch = -(-b_cnt // (NW * C))
    bp = NW * C * nch
    pad = bp - b_cnt

    pad_nodes = jnp.arange(pad, dtype=jnp.int32) % n_nodes
    pad_neigh = (jnp.arange(pad * s_cnt, dtype=jnp.int32)
                 % n_nodes).reshape(pad, s_cnt)
    nodes_p = jnp.concatenate([nodes.astype(jnp.int32), pad_nodes])
    neigh_p = jnp.concatenate(
        [neigh_idx.astype(jnp.int32), pad_neigh], axis=0)
    self_part = nodes_p.reshape(NW, nch, 1, C)
    neigh_part = neigh_p.reshape(NW, nch, C, s_cnt).transpose(0, 1, 3, 2)
    idx_all = jnp.concatenate([self_part, neigh_part], axis=2)
    idx_all = idx_all.reshape(NW, nch * (1 + s_cnt), C)

    mesh = plsc.VectorSubcoreMesh(core_axis_name="c", subcore_axis_name="s")
    sc = pl.kernel(
        functools.partial(_sc_body, nch, s_cnt, d),
        out_type=[
            jax.ShapeDtypeStruct((bp, d), jnp.float32),
            jax.ShapeDtypeStruct((bp, d), jnp.float32),
        ],
        mesh=mesh,
        scratch_types=[
            pltpu.VMEM((nch * (1 + s_cnt), C), jnp.int32),
            pltpu.VMEM((nch, C, d), jnp.float32),
            pltpu.VMEM((nch, C, d), jnp.float32),
            pltpu.SemaphoreType.DMA,
            pltpu.SemaphoreType.DMA((nch,)),
            pltpu.SemaphoreType.DMA((nch,)),
            pltpu.SemaphoreType.DMA((nch,)),
            pltpu.SemaphoreType.DMA((nch,)),
        ],
    )
    self_rows, nsum = sc(idx_all, features)

    bm = 2048
    out = pl.pallas_call(
        functools.partial(_tc_body, 1.0 / s_cnt, d),
        grid=(bp // bm,),
        in_specs=[
            pl.BlockSpec((bm, d), lambda i: (i, 0)),
            pl.BlockSpec((bm, d), lambda i: (i, 0)),
            pl.BlockSpec((d, 2 * d), lambda i: (0, 0)),
        ],
        out_specs=pl.BlockSpec((bm, d), lambda i: (i, 0)),
        out_shape=jax.ShapeDtypeStruct((b_cnt, d), jnp.float32),
    )(self_rows, nsum, W)
    return out
